# Initial kernel scaffold; baseline (speedup 1.0000x reference)
import jax
import jax.numpy as jnp
from jax.experimental import pallas as pl


def kernel(x, edge_index, edge_attr, W1, b1, W2, b2, Wl, bl):
    raise NotImplementedError("write your pallas kernel here")


# speedup vs baseline: 2.8981x; 2.8981x over previous
import jax
import jax.numpy as jnp
from jax.experimental import pallas as pl


def _head_kernel(h_ref, wl_ref, bl_ref, o_ref):
    o_ref[...] = (
        jnp.dot(h_ref[...], wl_ref[...], preferred_element_type=jnp.float32)
        + bl_ref[...]
    )


def kernel(x, edge_index, edge_attr, W1, b1, W2, b2, Wl, bl):
    n = x.shape[0]
    src = edge_index[0]
    dst = edge_index[1]
    deg = jax.ops.segment_sum(jnp.ones_like(dst, jnp.float32), dst, num_segments=n)
    deg = deg + 1.0
    dinv = jax.lax.rsqrt(deg)

    def layer(h, W, b):
        y = dinv[:, None] * (h @ W)
        acc = jax.ops.segment_sum(y[src], dst, num_segments=n)
        return jax.nn.relu(dinv[:, None] * (acc + y) + b)

    h = layer(x, W1, b1)
    h = layer(h, W2, b2)

    BM = 2000
    out = pl.pallas_call(
        _head_kernel,
        grid=(n // BM,),
        in_specs=[
            pl.BlockSpec((BM, h.shape[1]), lambda i: (i, 0)),
            pl.BlockSpec((h.shape[1], Wl.shape[1]), lambda i: (0, 0)),
            pl.BlockSpec((Wl.shape[1],), lambda i: (0,)),
        ],
        out_specs=pl.BlockSpec((BM, Wl.shape[1]), lambda i: (i, 0)),
        out_shape=jax.ShapeDtypeStruct((n, Wl.shape[1]), jnp.float32),
    )(h, Wl, bl)
    return out


# speedup vs baseline: 8.3488x; 2.8808x over previous
import functools

import jax
import jax.numpy as jnp
from jax import lax
from jax.experimental import pallas as pl
from jax.experimental.pallas import tpu as pltpu
from jax.experimental.pallas import tpu_sc as plsc

N = 10000
D = 128
H = 128
E = 320000

NC = 2
NS = 16
K = 128
G = 8
NG = 10
CHUNKS = NG * G
EPAD = NC * NS * CHUNKS * K
TRASH = N
NPAD = 10112
STRIPE = NPAD // NS
NPAD_D = 10240
STRIPE_D = NPAD_D // NS

_mesh = plsc.VectorSubcoreMesh(core_axis_name="c", subcore_axis_name="s")

def _zero_tile_rows(buf):
    z16 = jnp.zeros((16,), jnp.float32)

    @pl.loop(0, buf.shape[0])
    def _(r):
        @pl.loop(0, buf.shape[1] // 16)
        def _(c):
            buf[r, pl.ds(c * 16, 16)] = z16


@functools.partial(
    pl.kernel,
    out_type=jax.ShapeDtypeStruct((NC, NPAD_D), jnp.float32),
    mesh=_mesh,
    scratch_types=[
        pltpu.VMEM((CHUNKS, K), jnp.int32),
        pltpu.VMEM((K,), jnp.float32),
        pltpu.VMEM((STRIPE_D,), jnp.float32),
        pltpu.VMEM_SHARED((NPAD_D,), jnp.float32),
    ],
)
def _sc_deg(dsts_hbm, out_hbm, dst_v, ones_v, zb_v, acc_sh):
    cid = lax.axis_index("c")
    sid = lax.axis_index("s")
    pltpu.sync_copy(dsts_hbm.at[cid, sid], dst_v)

    one16 = jnp.ones((16,), jnp.float32)
    z16 = jnp.zeros((16,), jnp.float32)

    @pl.loop(0, K // 16)
    def _(i):
        ones_v[pl.ds(i * 16, 16)] = one16

    @pl.loop(0, STRIPE_D // 16)
    def _(i):
        zb_v[pl.ds(i * 16, 16)] = z16

    pltpu.sync_copy(zb_v, acc_sh.at[pl.ds(sid * STRIPE_D, STRIPE_D)])
    plsc.subcore_barrier()

    @pl.loop(0, CHUNKS)
    def _(j):
        pltpu.sync_copy(ones_v, acc_sh.at[dst_v.at[j]], add=True)

    plsc.subcore_barrier()
    pltpu.sync_copy(
        acc_sh.at[pl.ds(sid * STRIPE_D, STRIPE_D)],
        out_hbm.at[cid, pl.ds(sid * STRIPE_D, STRIPE_D)],
    )


@functools.partial(
    pl.kernel,
    out_type=jax.ShapeDtypeStruct((NC, NPAD, H), jnp.float32),
    mesh=_mesh,
    scratch_types=[
        pltpu.VMEM((G, 2, K), jnp.int32),
        pltpu.VMEM((G, 2, K), jnp.int32),
        pltpu.VMEM((K, H), jnp.float32),
        pltpu.VMEM((K, H), jnp.float32),
        pltpu.VMEM_SHARED((NPAD, H), jnp.float32),
        pltpu.SemaphoreType.DMA,
        pltpu.SemaphoreType.DMA,
        pltpu.SemaphoreType.DMA,
    ],
)
def _sc_agg(y_hbm, idx_hbm, out_hbm, ig_a, ig_b, rows_a, rows_b,
            acc_sh, sem_a, sem_b, sem_ig):
    cid = lax.axis_index("c")
    sid = lax.axis_index("s")

    _zero_tile_rows(rows_a)

    @pl.loop(0, STRIPE // K)
    def _(i):
        pltpu.sync_copy(rows_a, acc_sh.at[pl.ds(sid * STRIPE + i * K, K)])

    pltpu.sync_copy(
        rows_a.at[pl.ds(0, STRIPE % K)],
        acc_sh.at[pl.ds(sid * STRIPE + (STRIPE // K) * K, STRIPE % K)],
    )
    plsc.subcore_barrier()

    dummy_rows = y_hbm.at[pl.ds(0, K)]
    dummy_ig = idx_hbm.at[cid, sid, 0]
    sems = (sem_a, sem_b)

    pltpu.sync_copy(idx_hbm.at[cid, sid, 0], ig_a)
    pltpu.async_copy(y_hbm.at[ig_a.at[0, 0]], rows_a, sem_a)

    def group_body(g, cur, nxt):
        pltpu.async_copy(
            idx_hbm.at[cid, sid, jnp.minimum(g + 1, NG - 1)], nxt, sem_ig)
        bufs = (rows_a, rows_b)
        for i in range(G):
            if i == G - 1:
                pltpu.make_async_copy(dummy_ig, nxt, sem_ig).wait()
                nxt_idx = nxt.at[0, 0]
            else:
                nxt_idx = cur.at[i + 1, 0]
            buf, obuf = bufs[i % 2], bufs[(i + 1) % 2]
            pltpu.make_async_copy(dummy_rows, buf, sems[i % 2]).wait()
            pltpu.async_copy(y_hbm.at[nxt_idx], obuf, sems[(i + 1) % 2])
            pltpu.sync_copy(buf, acc_sh.at[cur.at[i, 1]], add=True)

    @pl.loop(0, NG // 2)
    def _(p):
        group_body(p * 2, ig_a, ig_b)
        group_body(p * 2 + 1, ig_b, ig_a)

    pltpu.make_async_copy(dummy_rows, rows_a, sem_a).wait()

    plsc.subcore_barrier()
    pltpu.sync_copy(
        acc_sh.at[pl.ds(sid * STRIPE, STRIPE)],
        out_hbm.at[cid, pl.ds(sid * STRIPE, STRIPE)],
    )


BM = 2000


def _tc_mm_kernel(x_ref, w_ref, o_ref):
    o_ref[...] = jnp.dot(x_ref[...], w_ref[...],
                         preferred_element_type=jnp.float32)


def _tc_scale_kernel(degt_ref, xw_ref, y_ref):
    d = degt_ref[...]
    dinv = lax.rsqrt(d[:, 0:1] + d[:, 1:2] + 1.0)
    y_ref[...] = xw_ref[...] * dinv


def _tc_layer_kernel(degt_ref, a0_ref, a1_ref, y_ref, b_ref, w_ref, o_ref):
    d = degt_ref[...]
    dinv = lax.rsqrt(d[:, 0:1] + d[:, 1:2] + 1.0)
    h = jnp.maximum(dinv * (a0_ref[...] + a1_ref[...] + y_ref[...])
                    + b_ref[...], 0.0)
    o_ref[...] = dinv * jnp.dot(h, w_ref[...],
                                preferred_element_type=jnp.float32)


def _tc_head_kernel(degt_ref, a0_ref, a1_ref, y_ref, b_ref, wl_ref, bl_ref,
                    o_ref):
    d = degt_ref[...]
    dinv = lax.rsqrt(d[:, 0:1] + d[:, 1:2] + 1.0)
    h = jnp.maximum(dinv * (a0_ref[...] + a1_ref[...] + y_ref[...])
                    + b_ref[...], 0.0)
    o_ref[...] = jnp.dot(h, wl_ref[...],
                         preferred_element_type=jnp.float32) + bl_ref[...]


def _tc_mm(x, w):
    return pl.pallas_call(
        _tc_mm_kernel,
        grid=(N // BM,),
        in_specs=[
            pl.BlockSpec((BM, D), lambda i: (i, 0)),
            pl.BlockSpec((D, H), lambda i: (0, 0)),
        ],
        out_specs=pl.BlockSpec((BM, H), lambda i: (i, 0)),
        out_shape=jax.ShapeDtypeStruct((N, H), jnp.float32),
    )(x, w)


def _tc_scale(degt, xw):
    return pl.pallas_call(
        _tc_scale_kernel,
        grid=(N // BM,),
        in_specs=[
            pl.BlockSpec((BM, 2), lambda i: (i, 0)),
            pl.BlockSpec((BM, H), lambda i: (i, 0)),
        ],
        out_specs=pl.BlockSpec((BM, H), lambda i: (i, 0)),
        out_shape=jax.ShapeDtypeStruct((N, H), jnp.float32),
    )(degt, xw)


def _tc_layer(degt, acc, y, b, w):
    return pl.pallas_call(
        _tc_layer_kernel,
        grid=(N // BM,),
        in_specs=[
            pl.BlockSpec((BM, 2), lambda i: (i, 0)),
            pl.BlockSpec((BM, H), lambda i: (i, 0)),
            pl.BlockSpec((BM, H), lambda i: (i, 0)),
            pl.BlockSpec((BM, H), lambda i: (i, 0)),
            pl.BlockSpec((1, H), lambda i: (0, 0)),
            pl.BlockSpec((H, H), lambda i: (0, 0)),
        ],
        out_specs=pl.BlockSpec((BM, H), lambda i: (i, 0)),
        out_shape=jax.ShapeDtypeStruct((N, H), jnp.float32),
    )(degt, acc[0, :N], acc[1, :N], y, b.reshape(1, H), w)


def _tc_head(degt, acc, y, b, wl, bl):
    no = wl.shape[1]
    return pl.pallas_call(
        _tc_head_kernel,
        grid=(N // BM,),
        in_specs=[
            pl.BlockSpec((BM, 2), lambda i: (i, 0)),
            pl.BlockSpec((BM, H), lambda i: (i, 0)),
            pl.BlockSpec((BM, H), lambda i: (i, 0)),
            pl.BlockSpec((BM, H), lambda i: (i, 0)),
            pl.BlockSpec((1, H), lambda i: (0, 0)),
            pl.BlockSpec((H, no), lambda i: (0, 0)),
            pl.BlockSpec((1, no), lambda i: (0, 0)),
        ],
        out_specs=pl.BlockSpec((BM, no), lambda i: (i, 0)),
        out_shape=jax.ShapeDtypeStruct((N, no), jnp.float32),
    )(degt, acc[0, :N], acc[1, :N], y, b.reshape(1, H), wl, bl.reshape(1, no))


def kernel(x, edge_index, edge_attr, W1, b1, W2, b2, Wl, bl):
    src = edge_index[0]
    dst = edge_index[1]
    pad = EPAD - E
    srcp = jnp.concatenate([src, jnp.zeros((pad,), jnp.int32)])
    srcp = srcp.reshape(NC, NS, CHUNKS, K)
    dstp = jnp.concatenate([dst, jnp.full((pad,), TRASH, jnp.int32)])
    dstp = dstp.reshape(NC, NS, CHUNKS, K)
    pairs = jnp.stack([srcp, dstp], axis=3)
    pairs = pairs.reshape(NC, NS, NG, G, 2, K)

    degp = _sc_deg(dstp)
    xw1 = _tc_mm(x, W1)
    degt = jnp.transpose(degp)[:N]

    y1 = _tc_scale(degt, xw1)
    acc1 = _sc_agg(y1, pairs)
    y2 = _tc_layer(degt, acc1, y1, b1, W2)
    acc2 = _sc_agg(y2, pairs)
    out = _tc_head(degt, acc2, y2, b2, Wl, bl)
    return out


# speedup vs baseline: 28.2213x; 3.3803x over previous
import functools

import jax
import jax.numpy as jnp
from jax import lax
from jax.experimental import pallas as pl
from jax.experimental.pallas import tpu as pltpu
from jax.experimental.pallas import tpu_sc as plsc

N = 10000
D = 128
H = 128
E = 320000

NC = 2
NS = 16
K = 128
G = 8
NG = 10
CHUNKS = NG * G
EPAD = NC * NS * CHUNKS * K
TRASH = N
NPAD = 10112
STRIPE = NPAD // NS
NPAD_D = 10240
STRIPE_D = NPAD_D // NS

_mesh = plsc.VectorSubcoreMesh(core_axis_name="c", subcore_axis_name="s")

def _zero_tile_rows(buf):
    z16 = jnp.zeros((16,), jnp.float32)

    @pl.loop(0, buf.shape[0])
    def _(r):
        @pl.loop(0, buf.shape[1] // 16)
        def _(c):
            buf[r, pl.ds(c * 16, 16)] = z16


@functools.partial(
    pl.kernel,
    out_type=jax.ShapeDtypeStruct((NC, NPAD_D), jnp.float32),
    mesh=_mesh,
    scratch_types=[
        pltpu.VMEM((CHUNKS, K), jnp.int32),
        pltpu.VMEM((K,), jnp.float32),
        pltpu.VMEM((STRIPE_D,), jnp.float32),
        pltpu.VMEM_SHARED((NPAD_D,), jnp.float32),
    ],
)
def _sc_deg(dsts_hbm, out_hbm, dst_v, ones_v, zb_v, acc_sh):
    cid = lax.axis_index("c")
    sid = lax.axis_index("s")
    pltpu.sync_copy(dsts_hbm.at[cid, sid], dst_v)

    one16 = jnp.ones((16,), jnp.float32)
    z16 = jnp.zeros((16,), jnp.float32)

    @pl.loop(0, K // 16)
    def _(i):
        ones_v[pl.ds(i * 16, 16)] = one16

    @pl.loop(0, STRIPE_D // 16)
    def _(i):
        zb_v[pl.ds(i * 16, 16)] = z16

    pltpu.sync_copy(zb_v, acc_sh.at[pl.ds(sid * STRIPE_D, STRIPE_D)])
    plsc.subcore_barrier()

    @pl.loop(0, CHUNKS)
    def _(j):
        pltpu.sync_copy(ones_v, acc_sh.at[dst_v.at[j]], add=True)

    plsc.subcore_barrier()
    pltpu.sync_copy(
        acc_sh.at[pl.ds(sid * STRIPE_D, STRIPE_D)],
        out_hbm.at[cid, pl.ds(sid * STRIPE_D, STRIPE_D)],
    )


@functools.partial(
    pl.kernel,
    out_type=jax.ShapeDtypeStruct((NC, NPAD, H), jnp.float32),
    mesh=_mesh,
    scratch_types=[
        pltpu.VMEM((G, 2, K), jnp.int32),
        pltpu.VMEM((G, 2, K), jnp.int32),
        pltpu.VMEM((K, H), jnp.float32),
        pltpu.VMEM((K, H), jnp.float32),
        pltpu.VMEM_SHARED((NPAD, H), jnp.float32),
        pltpu.SemaphoreType.DMA,
        pltpu.SemaphoreType.DMA,
        pltpu.SemaphoreType.DMA,
    ],
)
def _sc_agg(y_hbm, idx_hbm, out_hbm, ig_a, ig_b, rows_a, rows_b,
            acc_sh, sem_a, sem_b, sem_ig):
    cid = lax.axis_index("c")
    sid = lax.axis_index("s")

    _zero_tile_rows(rows_a)

    @pl.loop(0, STRIPE // K)
    def _(i):
        pltpu.sync_copy(rows_a, acc_sh.at[pl.ds(sid * STRIPE + i * K, K)])

    pltpu.sync_copy(
        rows_a.at[pl.ds(0, STRIPE % K)],
        acc_sh.at[pl.ds(sid * STRIPE + (STRIPE // K) * K, STRIPE % K)],
    )
    plsc.subcore_barrier()

    dummy_rows = y_hbm.at[pl.ds(0, K)]
    dummy_ig = idx_hbm.at[cid, sid, 0]
    sems = (sem_a, sem_b)

    pltpu.sync_copy(idx_hbm.at[cid, sid, 0], ig_a)
    pltpu.async_copy(y_hbm.at[ig_a.at[0, 0]], rows_a, sem_a)

    def group_body(g, cur, nxt):
        pltpu.async_copy(
            idx_hbm.at[cid, sid, jnp.minimum(g + 1, NG - 1)], nxt, sem_ig)
        bufs = (rows_a, rows_b)
        for i in range(G):
            if i == G - 1:
                pltpu.make_async_copy(dummy_ig, nxt, sem_ig).wait()
                nxt_idx = nxt.at[0, 0]
            else:
                nxt_idx = cur.at[i + 1, 0]
            buf, obuf = bufs[i % 2], bufs[(i + 1) % 2]
            pltpu.make_async_copy(dummy_rows, buf, sems[i % 2]).wait()
            pltpu.async_copy(y_hbm.at[nxt_idx], obuf, sems[(i + 1) % 2])
            pltpu.sync_copy(buf, acc_sh.at[cur.at[i, 1]], add=True)

    @pl.loop(0, NG // 2)
    def _(p):
        group_body(p * 2, ig_a, ig_b)
        group_body(p * 2 + 1, ig_b, ig_a)

    pltpu.make_async_copy(dummy_rows, rows_a, sem_a).wait()

    plsc.subcore_barrier()
    pltpu.sync_copy(
        acc_sh.at[pl.ds(sid * STRIPE, STRIPE)],
        out_hbm.at[cid, pl.ds(sid * STRIPE, STRIPE)],
    )


BM = 2000


def _tc_mm_kernel(x_ref, w_ref, o_ref):
    o_ref[...] = jnp.dot(x_ref[...], w_ref[...],
                         preferred_element_type=jnp.float32)


def _tc_scale_kernel(degt_ref, xw_ref, y_ref):
    d = degt_ref[...]
    dinv = lax.rsqrt(d[:, 0:1] + d[:, 1:2] + 1.0)
    y_ref[...] = xw_ref[...] * dinv


def _tc_layer_kernel(degt_ref, a0_ref, a1_ref, y_ref, b_ref, w_ref, o_ref):
    d = degt_ref[...]
    dinv = lax.rsqrt(d[:, 0:1] + d[:, 1:2] + 1.0)
    h = jnp.maximum(dinv * (a0_ref[...] + a1_ref[...] + y_ref[...])
                    + b_ref[...], 0.0)
    o_ref[...] = dinv * jnp.dot(h, w_ref[...],
                                preferred_element_type=jnp.float32)


def _tc_head_kernel(degt_ref, a0_ref, a1_ref, y_ref, b_ref, wl_ref, bl_ref,
                    o_ref):
    d = degt_ref[...]
    dinv = lax.rsqrt(d[:, 0:1] + d[:, 1:2] + 1.0)
    h = jnp.maximum(dinv * (a0_ref[...] + a1_ref[...] + y_ref[...])
                    + b_ref[...], 0.0)
    o_ref[...] = jnp.dot(h, wl_ref[...],
                         preferred_element_type=jnp.float32) + bl_ref[...]


def _tc_mm(x, w):
    return pl.pallas_call(
        _tc_mm_kernel,
        grid=(N // BM,),
        in_specs=[
            pl.BlockSpec((BM, D), lambda i: (i, 0)),
            pl.BlockSpec((D, H), lambda i: (0, 0)),
        ],
        out_specs=pl.BlockSpec((BM, H), lambda i: (i, 0)),
        out_shape=jax.ShapeDtypeStruct((N, H), jnp.float32),
    )(x, w)


def _tc_scale(degt, xw):
    return pl.pallas_call(
        _tc_scale_kernel,
        grid=(N // BM,),
        in_specs=[
            pl.BlockSpec((BM, 2), lambda i: (i, 0)),
            pl.BlockSpec((BM, H), lambda i: (i, 0)),
        ],
        out_specs=pl.BlockSpec((BM, H), lambda i: (i, 0)),
        out_shape=jax.ShapeDtypeStruct((N, H), jnp.float32),
    )(degt, xw)


def _tc_layer(degt, acc, y, b, w):
    return pl.pallas_call(
        _tc_layer_kernel,
        grid=(N // BM,),
        in_specs=[
            pl.BlockSpec((BM, 2), lambda i: (i, 0)),
            pl.BlockSpec((BM, H), lambda i: (i, 0)),
            pl.BlockSpec((BM, H), lambda i: (i, 0)),
            pl.BlockSpec((BM, H), lambda i: (i, 0)),
            pl.BlockSpec((1, H), lambda i: (0, 0)),
            pl.BlockSpec((H, H), lambda i: (0, 0)),
        ],
        out_specs=pl.BlockSpec((BM, H), lambda i: (i, 0)),
        out_shape=jax.ShapeDtypeStruct((N, H), jnp.float32),
    )(degt, acc[0, :N], acc[1, :N], y, b.reshape(1, H), w)


def _tc_head(degt, acc, y, b, wl, bl):
    no = wl.shape[1]
    return pl.pallas_call(
        _tc_head_kernel,
        grid=(N // BM,),
        in_specs=[
            pl.BlockSpec((BM, 2), lambda i: (i, 0)),
            pl.BlockSpec((BM, H), lambda i: (i, 0)),
            pl.BlockSpec((BM, H), lambda i: (i, 0)),
            pl.BlockSpec((BM, H), lambda i: (i, 0)),
            pl.BlockSpec((1, H), lambda i: (0, 0)),
            pl.BlockSpec((H, no), lambda i: (0, 0)),
            pl.BlockSpec((1, no), lambda i: (0, 0)),
        ],
        out_specs=pl.BlockSpec((BM, no), lambda i: (i, 0)),
        out_shape=jax.ShapeDtypeStruct((N, no), jnp.float32),
    )(degt, acc[0, :N], acc[1, :N], y, b.reshape(1, H), wl, bl.reshape(1, no))


def kernel(x, edge_index, edge_attr, W1, b1, W2, b2, Wl, bl):
    src = edge_index[0]
    dst = edge_index[1]
    pad = EPAD - E
    ar = jnp.arange(pad, dtype=jnp.int32)
    srcp = jnp.concatenate([src, ar % N])
    srcp = srcp.reshape(NC, NS, CHUNKS, K)
    dstp = jnp.concatenate([dst, TRASH + ar % (NPAD - TRASH)])
    dstp = dstp.reshape(NC, NS, CHUNKS, K)
    pairs = jnp.stack([srcp, dstp], axis=3)
    pairs = pairs.reshape(NC, NS, NG, G, 2, K)

    degp = _sc_deg(dstp)
    xw1 = _tc_mm(x, W1)
    degt = jnp.transpose(degp)[:N]

    y1 = _tc_scale(degt, xw1)
    acc1 = _sc_agg(y1, pairs)
    y2 = _tc_layer(degt, acc1, y1, b1, W2)
    acc2 = _sc_agg(y2, pairs)
    out = _tc_head(degt, acc2, y2, b2, Wl, bl)
    return out


# speedup vs baseline: 28.2412x; 1.0007x over previous
import functools

import jax
import jax.numpy as jnp
from jax import lax
from jax.experimental import pallas as pl
from jax.experimental.pallas import tpu as pltpu
from jax.experimental.pallas import tpu_sc as plsc

N = 10000
D = 128
H = 128
E = 320000

NC = 2
NS = 16
K = 128
G = 8
NG = 10
CHUNKS = NG * G
EPAD = NC * NS * CHUNKS * K
TRASH = N
NPAD = 10112
STRIPE = NPAD // NS
NPAD_D = 10240
STRIPE_D = NPAD_D // NS

_mesh = plsc.VectorSubcoreMesh(core_axis_name="c", subcore_axis_name="s")

def _zero_tile_rows(buf):
    z16 = jnp.zeros((16,), jnp.float32)

    @pl.loop(0, buf.shape[0])
    def _(r):
        @pl.loop(0, buf.shape[1] // 16)
        def _(c):
            buf[r, pl.ds(c * 16, 16)] = z16


@functools.partial(
    pl.kernel,
    out_type=jax.ShapeDtypeStruct((NC, NPAD_D), jnp.float32),
    mesh=_mesh,
    scratch_types=[
        pltpu.VMEM((CHUNKS, K), jnp.int32),
        pltpu.VMEM((K,), jnp.float32),
        pltpu.VMEM((STRIPE_D,), jnp.float32),
        pltpu.VMEM_SHARED((NPAD_D,), jnp.float32),
    ],
)
def _sc_deg(dsts_hbm, out_hbm, dst_v, ones_v, zb_v, acc_sh):
    cid = lax.axis_index("c")
    sid = lax.axis_index("s")
    pltpu.sync_copy(dsts_hbm.at[cid, sid], dst_v)

    one16 = jnp.ones((16,), jnp.float32)
    z16 = jnp.zeros((16,), jnp.float32)

    @pl.loop(0, K // 16)
    def _(i):
        ones_v[pl.ds(i * 16, 16)] = one16

    @pl.loop(0, STRIPE_D // 16)
    def _(i):
        zb_v[pl.ds(i * 16, 16)] = z16

    pltpu.sync_copy(zb_v, acc_sh.at[pl.ds(sid * STRIPE_D, STRIPE_D)])
    plsc.subcore_barrier()

    @pl.loop(0, CHUNKS)
    def _(j):
        pltpu.sync_copy(ones_v, acc_sh.at[dst_v.at[j]], add=True)

    plsc.subcore_barrier()
    pltpu.sync_copy(
        acc_sh.at[pl.ds(sid * STRIPE_D, STRIPE_D)],
        out_hbm.at[cid, pl.ds(sid * STRIPE_D, STRIPE_D)],
    )


@functools.partial(
    pl.kernel,
    out_type=jax.ShapeDtypeStruct((NC, NPAD, H), jnp.float32),
    mesh=_mesh,
    scratch_types=[
        pltpu.VMEM((G, 2, K), jnp.int32),
        pltpu.VMEM((G, 2, K), jnp.int32),
        pltpu.VMEM((K, H), jnp.float32),
        pltpu.VMEM((K, H), jnp.float32),
        pltpu.VMEM_SHARED((NPAD, H), jnp.float32),
        pltpu.SemaphoreType.DMA,
        pltpu.SemaphoreType.DMA,
        pltpu.SemaphoreType.DMA,
    ],
)
def _sc_agg(y_hbm, idx_hbm, out_hbm, ig_a, ig_b, rows_a, rows_b,
            acc_sh, sem_a, sem_b, sem_ig):
    cid = lax.axis_index("c")
    sid = lax.axis_index("s")

    _zero_tile_rows(rows_a)

    @pl.loop(0, STRIPE // K)
    def _(i):
        pltpu.sync_copy(rows_a, acc_sh.at[pl.ds(sid * STRIPE + i * K, K)])

    pltpu.sync_copy(
        rows_a.at[pl.ds(0, STRIPE % K)],
        acc_sh.at[pl.ds(sid * STRIPE + (STRIPE // K) * K, STRIPE % K)],
    )
    plsc.subcore_barrier()

    dummy_rows = y_hbm.at[pl.ds(0, K)]
    dummy_ig = idx_hbm.at[cid, sid, 0]
    sems = (sem_a, sem_b)

    pltpu.sync_copy(idx_hbm.at[cid, sid, 0], ig_a)
    pltpu.async_copy(y_hbm.at[ig_a.at[0, 0]], rows_a, sem_a)

    def group_body(g, cur, nxt):
        pltpu.async_copy(
            idx_hbm.at[cid, sid, jnp.minimum(g + 1, NG - 1)], nxt, sem_ig)
        bufs = (rows_a, rows_b)
        for i in range(G):
            if i == G - 1:
                pltpu.make_async_copy(dummy_ig, nxt, sem_ig).wait()
                nxt_idx = nxt.at[0, 0]
            else:
                nxt_idx = cur.at[i + 1, 0]
            buf, obuf = bufs[i % 2], bufs[(i + 1) % 2]
            pltpu.make_async_copy(dummy_rows, buf, sems[i % 2]).wait()
            pltpu.async_copy(y_hbm.at[nxt_idx], obuf, sems[(i + 1) % 2])
            pltpu.sync_copy(buf, acc_sh.at[cur.at[i, 1]], add=True)

    @pl.loop(0, NG // 2)
    def _(p):
        group_body(p * 2, ig_a, ig_b)
        group_body(p * 2 + 1, ig_b, ig_a)

    pltpu.make_async_copy(dummy_rows, rows_a, sem_a).wait()

    plsc.subcore_barrier()
    pltpu.sync_copy(
        acc_sh.at[pl.ds(sid * STRIPE, STRIPE)],
        out_hbm.at[cid, pl.ds(sid * STRIPE, STRIPE)],
    )


BM = 2000


def _dinv_col(degt):
    return lax.rsqrt(degt[:, 0:1] + degt[:, 1:2] + 1.0)


def _tc_mms_kernel(degt_ref, x_ref, w_ref, o_ref):
    dinv = _dinv_col(degt_ref[...])
    o_ref[...] = dinv * jnp.dot(x_ref[...], w_ref[...],
                                preferred_element_type=jnp.float32)


def _tc_layer_kernel(degt_ref, a0_ref, a1_ref, y_ref, b_ref, w_ref, o_ref):
    dinv = _dinv_col(degt_ref[...])
    h = jnp.maximum(dinv * (a0_ref[...] + a1_ref[...] + y_ref[...])
                    + b_ref[...], 0.0)
    o_ref[...] = dinv * jnp.dot(h, w_ref[...],
                                preferred_element_type=jnp.float32)


def _tc_head_kernel(degt_ref, a0_ref, a1_ref, y_ref, b_ref, wl_ref, bl_ref,
                    o_ref):
    dinv = _dinv_col(degt_ref[...])
    h = jnp.maximum(dinv * (a0_ref[...] + a1_ref[...] + y_ref[...])
                    + b_ref[...], 0.0)
    o_ref[...] = jnp.dot(h, wl_ref[...],
                         preferred_element_type=jnp.float32) + bl_ref[...]


def _tc_mms(degt, x, w):
    return pl.pallas_call(
        _tc_mms_kernel,
        grid=(N // BM,),
        in_specs=[
            pl.BlockSpec((BM, 2), lambda i: (i, 0)),
            pl.BlockSpec((BM, D), lambda i: (i, 0)),
            pl.BlockSpec((D, H), lambda i: (0, 0)),
        ],
        out_specs=pl.BlockSpec((BM, H), lambda i: (i, 0)),
        out_shape=jax.ShapeDtypeStruct((N, H), jnp.float32),
    )(degt, x, w)


def _tc_layer(degt, acc, y, b, w):
    return pl.pallas_call(
        _tc_layer_kernel,
        grid=(N // BM,),
        in_specs=[
            pl.BlockSpec((BM, 2), lambda i: (i, 0)),
            pl.BlockSpec((BM, H), lambda i: (i, 0)),
            pl.BlockSpec((BM, H), lambda i: (i, 0)),
            pl.BlockSpec((BM, H), lambda i: (i, 0)),
            pl.BlockSpec((1, H), lambda i: (0, 0)),
            pl.BlockSpec((H, H), lambda i: (0, 0)),
        ],
        out_specs=pl.BlockSpec((BM, H), lambda i: (i, 0)),
        out_shape=jax.ShapeDtypeStruct((N, H), jnp.float32),
    )(degt, acc[0, :N], acc[1, :N], y, b.reshape(1, H), w)


def _tc_head(degt, acc, y, b, wl, bl):
    no = wl.shape[1]
    return pl.pallas_call(
        _tc_head_kernel,
        grid=(N // BM,),
        in_specs=[
            pl.BlockSpec((BM, 2), lambda i: (i, 0)),
            pl.BlockSpec((BM, H), lambda i: (i, 0)),
            pl.BlockSpec((BM, H), lambda i: (i, 0)),
            pl.BlockSpec((BM, H), lambda i: (i, 0)),
            pl.BlockSpec((1, H), lambda i: (0, 0)),
            pl.BlockSpec((H, no), lambda i: (0, 0)),
            pl.BlockSpec((1, no), lambda i: (0, 0)),
        ],
        out_specs=pl.BlockSpec((BM, no), lambda i: (i, 0)),
        out_shape=jax.ShapeDtypeStruct((N, no), jnp.float32),
    )(degt, acc[0, :N], acc[1, :N], y, b.reshape(1, H), wl, bl.reshape(1, no))


def kernel(x, edge_index, edge_attr, W1, b1, W2, b2, Wl, bl):
    src = edge_index[0]
    dst = edge_index[1]
    pad = EPAD - E
    ar = jnp.arange(pad, dtype=jnp.int32)
    srcp = jnp.concatenate([src, ar % N])
    srcp = srcp.reshape(NC, NS, CHUNKS, K)
    dstp = jnp.concatenate([dst, TRASH + ar % (NPAD - TRASH)])
    dstp = dstp.reshape(NC, NS, CHUNKS, K)
    pairs = jnp.stack([srcp, dstp], axis=3)
    pairs = pairs.reshape(NC, NS, NG, G, 2, K)

    degp = _sc_deg(dstp)
    degt = jnp.transpose(degp)[:N]
    y1 = _tc_mms(degt, x, W1)
    acc1 = _sc_agg(y1, pairs)
    y2 = _tc_layer(degt, acc1, y1, b1, W2)
    acc2 = _sc_agg(y2, pairs)
    out = _tc_head(degt, acc2, y2, b2, Wl, bl)
    return out


# speedup vs baseline: 29.9380x; 1.0601x over previous
import functools

import jax
import jax.numpy as jnp
from jax import lax
from jax.experimental import pallas as pl
from jax.experimental.pallas import tpu as pltpu
from jax.experimental.pallas import tpu_sc as plsc

N = 10000
D = 128
H = 128
E = 320000

NC = 2
NS = 16
K = 128
G = 8
NG = 10
CHUNKS = NG * G
EPAD = NC * NS * CHUNKS * K
TRASH = N
NPAD = 10112
STRIPE = NPAD // NS
NPAD_D = 10240
STRIPE_D = NPAD_D // NS

_mesh = plsc.VectorSubcoreMesh(core_axis_name="c", subcore_axis_name="s")

def _zero_tile_rows(buf):
    z16 = jnp.zeros((16,), jnp.float32)

    @pl.loop(0, buf.shape[0])
    def _(r):
        @pl.loop(0, buf.shape[1] // 16)
        def _(c):
            buf[r, pl.ds(c * 16, 16)] = z16


@functools.partial(
    pl.kernel,
    out_type=jax.ShapeDtypeStruct((NC, NPAD_D), jnp.float32),
    mesh=_mesh,
    scratch_types=[
        pltpu.VMEM((CHUNKS, K), jnp.int32),
        pltpu.VMEM((K,), jnp.float32),
        pltpu.VMEM((STRIPE_D,), jnp.float32),
        pltpu.VMEM_SHARED((NPAD_D,), jnp.float32),
    ],
)
def _sc_deg(dsts_hbm, out_hbm, dst_v, ones_v, zb_v, acc_sh):
    cid = lax.axis_index("c")
    sid = lax.axis_index("s")
    pltpu.sync_copy(dsts_hbm.at[cid, sid], dst_v)

    one16 = jnp.ones((16,), jnp.float32)
    z16 = jnp.zeros((16,), jnp.float32)

    @pl.loop(0, K // 16)
    def _(i):
        ones_v[pl.ds(i * 16, 16)] = one16

    @pl.loop(0, STRIPE_D // 16)
    def _(i):
        zb_v[pl.ds(i * 16, 16)] = z16

    pltpu.sync_copy(zb_v, acc_sh.at[pl.ds(sid * STRIPE_D, STRIPE_D)])
    plsc.subcore_barrier()

    @pl.loop(0, CHUNKS)
    def _(j):
        pltpu.sync_copy(ones_v, acc_sh.at[dst_v.at[j]], add=True)

    plsc.subcore_barrier()
    pltpu.sync_copy(
        acc_sh.at[pl.ds(sid * STRIPE_D, STRIPE_D)],
        out_hbm.at[cid, pl.ds(sid * STRIPE_D, STRIPE_D)],
    )


@functools.partial(
    pl.kernel,
    out_type=jax.ShapeDtypeStruct((NC, NPAD, H), jnp.float32),
    mesh=_mesh,
    scratch_types=[
        pltpu.VMEM((G, 2, K), jnp.int32),
        pltpu.VMEM((G, 2, K), jnp.int32),
        pltpu.VMEM((K, H), jnp.float32),
        pltpu.VMEM((K, H), jnp.float32),
        pltpu.VMEM_SHARED((NPAD, H), jnp.float32),
        pltpu.SemaphoreType.DMA,
        pltpu.SemaphoreType.DMA,
        pltpu.SemaphoreType.DMA,
    ],
)
def _sc_agg(y_hbm, idx_hbm, out_hbm, ig_a, ig_b, rows_a, rows_b,
            acc_sh, sem_a, sem_b, sem_ig):
    cid = lax.axis_index("c")
    sid = lax.axis_index("s")

    _zero_tile_rows(rows_a)

    @pl.loop(0, STRIPE // K)
    def _(i):
        pltpu.sync_copy(rows_a, acc_sh.at[pl.ds(sid * STRIPE + i * K, K)])

    pltpu.sync_copy(
        rows_a.at[pl.ds(0, STRIPE % K)],
        acc_sh.at[pl.ds(sid * STRIPE + (STRIPE // K) * K, STRIPE % K)],
    )
    plsc.subcore_barrier()

    dummy_rows = y_hbm.at[pl.ds(0, K)]
    dummy_ig = idx_hbm.at[cid, sid, 0]
    sems = (sem_a, sem_b)

    pltpu.sync_copy(idx_hbm.at[cid, sid, 0], ig_a)
    pltpu.async_copy(y_hbm.at[ig_a.at[0, 0]], rows_a, sem_a)

    def group_body(g, cur, nxt):
        pltpu.async_copy(
            idx_hbm.at[cid, sid, jnp.minimum(g + 1, NG - 1)], nxt, sem_ig)
        bufs = (rows_a, rows_b)
        for i in range(G):
            if i == G - 1:
                pltpu.make_async_copy(dummy_ig, nxt, sem_ig).wait()
                nxt_idx = nxt.at[0, 0]
            else:
                nxt_idx = cur.at[i + 1, 0]
            buf, obuf = bufs[i % 2], bufs[(i + 1) % 2]
            pltpu.make_async_copy(dummy_rows, buf, sems[i % 2]).wait()
            pltpu.async_copy(y_hbm.at[nxt_idx], obuf, sems[(i + 1) % 2])
            pltpu.sync_copy(buf, acc_sh.at[cur.at[i, 1]], add=True)

    @pl.loop(0, NG // 2)
    def _(p):
        group_body(p * 2, ig_a, ig_b)
        group_body(p * 2 + 1, ig_b, ig_a)

    pltpu.make_async_copy(dummy_rows, rows_a, sem_a).wait()

    plsc.subcore_barrier()
    pltpu.sync_copy(
        acc_sh.at[pl.ds(sid * STRIPE, STRIPE)],
        out_hbm.at[cid, pl.ds(sid * STRIPE, STRIPE)],
    )


BM = 2048
GRID = (N + BM - 1) // BM


def _tc_mm_kernel(x_ref, w_ref, o_ref):
    o_ref[...] = jnp.dot(x_ref[...], w_ref[...],
                         preferred_element_type=jnp.float32)


def _tc_scale_kernel(degp_ref, z_ref, y_ref, dinv_ref):
    d = degp_ref[...]
    dv = lax.rsqrt(d[0:1, :] + d[1:2, :] + 1.0)
    dcol = jnp.transpose(dv, (1, 0))
    dinv_ref[...] = dcol
    y_ref[...] = dcol * z_ref[...]


def _tc_layer_kernel(dinv_ref, a0_ref, a1_ref, y_ref, b_ref, w_ref, o_ref):
    dinv = dinv_ref[...]
    h = jnp.maximum(dinv * (a0_ref[0] + a1_ref[0] + y_ref[...])
                    + b_ref[...], 0.0)
    o_ref[...] = dinv * jnp.dot(h, w_ref[...],
                                preferred_element_type=jnp.float32)


def _tc_head_kernel(dinv_ref, a0_ref, a1_ref, y_ref, b_ref, wl_ref, bl_ref,
                    o_ref):
    dinv = dinv_ref[...]
    h = jnp.maximum(dinv * (a0_ref[0] + a1_ref[0] + y_ref[...])
                    + b_ref[...], 0.0)
    o_ref[...] = jnp.dot(h, wl_ref[...],
                         preferred_element_type=jnp.float32) + bl_ref[...]


def _tc_mm(x, w):
    return pl.pallas_call(
        _tc_mm_kernel,
        grid=(GRID,),
        in_specs=[
            pl.BlockSpec((BM, D), lambda i: (i, 0)),
            pl.BlockSpec((D, H), lambda i: (0, 0)),
        ],
        out_specs=pl.BlockSpec((BM, H), lambda i: (i, 0)),
        out_shape=jax.ShapeDtypeStruct((N, H), jnp.float32),
    )(x, w)


def _tc_scale(degp, z):
    return pl.pallas_call(
        _tc_scale_kernel,
        grid=(GRID,),
        in_specs=[
            pl.BlockSpec((2, BM), lambda i: (0, i)),
            pl.BlockSpec((BM, H), lambda i: (i, 0)),
        ],
        out_specs=[
            pl.BlockSpec((BM, H), lambda i: (i, 0)),
            pl.BlockSpec((BM, 1), lambda i: (i, 0)),
        ],
        out_shape=[
            jax.ShapeDtypeStruct((N, H), jnp.float32),
            jax.ShapeDtypeStruct((N, 1), jnp.float32),
        ],
    )(degp, z)


def _acc_specs():
    return [
        pl.BlockSpec((1, BM, H), lambda i: (0, i, 0)),
        pl.BlockSpec((1, BM, H), lambda i: (1, i, 0)),
    ]


def _tc_layer(dinv, acc, y, b, w):
    return pl.pallas_call(
        _tc_layer_kernel,
        grid=(GRID,),
        in_specs=[pl.BlockSpec((BM, 1), lambda i: (i, 0))] + _acc_specs() + [
            pl.BlockSpec((BM, H), lambda i: (i, 0)),
            pl.BlockSpec((1, H), lambda i: (0, 0)),
            pl.BlockSpec((H, H), lambda i: (0, 0)),
        ],
        out_specs=pl.BlockSpec((BM, H), lambda i: (i, 0)),
        out_shape=jax.ShapeDtypeStruct((N, H), jnp.float32),
    )(dinv, acc, acc, y, b.reshape(1, H), w)


def _tc_head(dinv, acc, y, b, wl, bl):
    no = wl.shape[1]
    return pl.pallas_call(
        _tc_head_kernel,
        grid=(GRID,),
        in_specs=[pl.BlockSpec((BM, 1), lambda i: (i, 0))] + _acc_specs() + [
            pl.BlockSpec((BM, H), lambda i: (i, 0)),
            pl.BlockSpec((1, H), lambda i: (0, 0)),
            pl.BlockSpec((H, no), lambda i: (0, 0)),
            pl.BlockSpec((1, no), lambda i: (0, 0)),
        ],
        out_specs=pl.BlockSpec((BM, no), lambda i: (i, 0)),
        out_shape=jax.ShapeDtypeStruct((N, no), jnp.float32),
    )(dinv, acc, acc, y, b.reshape(1, H), wl, bl.reshape(1, no))


def kernel(x, edge_index, edge_attr, W1, b1, W2, b2, Wl, bl):
    src = edge_index[0]
    dst = edge_index[1]
    pad = EPAD - E
    ar = jnp.arange(pad, dtype=jnp.int32)
    srcp = jnp.concatenate([src, ar % N])
    srcp = srcp.reshape(NC, NS, CHUNKS, K)
    dstp = jnp.concatenate([dst, TRASH + ar % (NPAD - TRASH)])
    dstp = dstp.reshape(NC, NS, CHUNKS, K)
    pairs = jnp.stack([srcp, dstp], axis=3)
    pairs = pairs.reshape(NC, NS, NG, G, 2, K)

    degp = _sc_deg(dstp)
    z1 = _tc_mm(x, W1)
    y1, dinv = _tc_scale(degp, z1)
    acc1 = _sc_agg(y1, pairs)
    y2 = _tc_layer(dinv, acc1, y1, b1, W2)
    acc2 = _sc_agg(y2, pairs)
    out = _tc_head(dinv, acc2, y2, b2, Wl, bl)
    return out


# speedup vs baseline: 35.1585x; 1.1744x over previous
import functools

import jax
import jax.numpy as jnp
from jax import lax
from jax.experimental import pallas as pl
from jax.experimental.pallas import tpu as pltpu
from jax.experimental.pallas import tpu_sc as plsc

N = 10000
D = 128
H = 128
E = 320000

NC = 2
NS = 16
K = 96
G = 9
NG = 12
CHUNKS = NG * G
EPAD = NC * NS * CHUNKS * K
TRASH = N
NPAD = 10112
STRIPE = NPAD // NS
NPAD_D = 10240
STRIPE_D = NPAD_D // NS

_mesh = plsc.VectorSubcoreMesh(core_axis_name="c", subcore_axis_name="s")

def _zero_tile_rows(buf):
    z16 = jnp.zeros((16,), jnp.float32)

    @pl.loop(0, buf.shape[0])
    def _(r):
        @pl.loop(0, buf.shape[1] // 16)
        def _(c):
            buf[r, pl.ds(c * 16, 16)] = z16


@functools.partial(
    pl.kernel,
    out_type=jax.ShapeDtypeStruct((NC, NPAD_D), jnp.float32),
    mesh=_mesh,
    scratch_types=[
        pltpu.VMEM((CHUNKS, K), jnp.int32),
        pltpu.VMEM((K,), jnp.float32),
        pltpu.VMEM((STRIPE_D,), jnp.float32),
        pltpu.VMEM_SHARED((NPAD_D,), jnp.float32),
    ],
)
def _sc_deg(dsts_hbm, out_hbm, dst_v, ones_v, zb_v, acc_sh):
    cid = lax.axis_index("c")
    sid = lax.axis_index("s")
    pltpu.sync_copy(dsts_hbm.at[cid, sid], dst_v)

    one16 = jnp.ones((16,), jnp.float32)
    z16 = jnp.zeros((16,), jnp.float32)

    @pl.loop(0, K // 16)
    def _(i):
        ones_v[pl.ds(i * 16, 16)] = one16

    @pl.loop(0, STRIPE_D // 16)
    def _(i):
        zb_v[pl.ds(i * 16, 16)] = z16

    pltpu.sync_copy(zb_v, acc_sh.at[pl.ds(sid * STRIPE_D, STRIPE_D)])
    plsc.subcore_barrier()

    @pl.loop(0, CHUNKS)
    def _(j):
        pltpu.sync_copy(ones_v, acc_sh.at[dst_v.at[j]], add=True)

    plsc.subcore_barrier()
    pltpu.sync_copy(
        acc_sh.at[pl.ds(sid * STRIPE_D, STRIPE_D)],
        out_hbm.at[cid, pl.ds(sid * STRIPE_D, STRIPE_D)],
    )


@functools.partial(
    pl.kernel,
    out_type=jax.ShapeDtypeStruct((NC, NPAD, H), jnp.float32),
    mesh=_mesh,
    scratch_types=[
        pltpu.VMEM((G, 2, K), jnp.int32),
        pltpu.VMEM((G, 2, K), jnp.int32),
        pltpu.VMEM((K, H), jnp.float32),
        pltpu.VMEM((K, H), jnp.float32),
        pltpu.VMEM((K, H), jnp.float32),
        pltpu.VMEM_SHARED((NPAD, H), jnp.float32),
        pltpu.SemaphoreType.DMA,
        pltpu.SemaphoreType.DMA,
        pltpu.SemaphoreType.DMA,
        pltpu.SemaphoreType.DMA,
    ],
)
def _sc_agg(y_hbm, idx_hbm, out_hbm, ig_a, ig_b, rows_a, rows_b, rows_c,
            acc_sh, sem_a, sem_b, sem_c, sem_ig):
    cid = lax.axis_index("c")
    sid = lax.axis_index("s")

    _zero_tile_rows(rows_a)

    @pl.loop(0, STRIPE // K)
    def _(i):
        pltpu.sync_copy(rows_a, acc_sh.at[pl.ds(sid * STRIPE + i * K, K)])

    pltpu.sync_copy(
        rows_a.at[pl.ds(0, STRIPE % K)],
        acc_sh.at[pl.ds(sid * STRIPE + (STRIPE // K) * K, STRIPE % K)],
    )
    plsc.subcore_barrier()

    dummy_rows = y_hbm.at[pl.ds(0, K)]
    dummy_ig = idx_hbm.at[cid, sid, 0]
    bufs = (rows_a, rows_b, rows_c)
    sems = (sem_a, sem_b, sem_c)

    pltpu.sync_copy(idx_hbm.at[cid, sid, 0], ig_a)
    pltpu.async_copy(y_hbm.at[ig_a.at[0, 0]], rows_a, sem_a)
    pltpu.async_copy(y_hbm.at[ig_a.at[1, 0]], rows_b, sem_b)

    def group_body(g, cur, nxt):
        pltpu.async_copy(
            idx_hbm.at[cid, sid, jnp.minimum(g + 1, NG - 1)], nxt, sem_ig)
        for i in range(G):
            if i == G - 2:
                pltpu.make_async_copy(dummy_ig, nxt, sem_ig).wait()
            nxt_idx = cur.at[i + 2, 0] if i + 2 < G else nxt.at[i + 2 - G, 0]
            buf, fbuf = bufs[i % 3], bufs[(i + 2) % 3]
            pltpu.make_async_copy(dummy_rows, buf, sems[i % 3]).wait()
            pltpu.async_copy(y_hbm.at[nxt_idx], fbuf, sems[(i + 2) % 3])
            pltpu.sync_copy(buf, acc_sh.at[cur.at[i, 1]], add=True)

    @pl.loop(0, NG // 2)
    def _(p):
        group_body(p * 2, ig_a, ig_b)
        group_body(p * 2 + 1, ig_b, ig_a)

    pltpu.make_async_copy(dummy_rows, rows_a, sem_a).wait()
    pltpu.make_async_copy(dummy_rows, rows_b, sem_b).wait()

    plsc.subcore_barrier()
    pltpu.sync_copy(
        acc_sh.at[pl.ds(sid * STRIPE, STRIPE)],
        out_hbm.at[cid, pl.ds(sid * STRIPE, STRIPE)],
    )


BM = 2048
GRID = (N + BM - 1) // BM


def _tc_mm_kernel(x_ref, w_ref, o_ref):
    o_ref[...] = jnp.dot(x_ref[...], w_ref[...],
                         preferred_element_type=jnp.float32)


def _tc_scale_kernel(degp_ref, z_ref, y_ref, dinv_ref):
    d = degp_ref[...]
    dv = lax.rsqrt(d[0:1, :] + d[1:2, :] + 1.0)
    dcol = jnp.transpose(dv, (1, 0))
    dinv_ref[...] = dcol
    y_ref[...] = dcol * z_ref[...]


def _tc_layer_kernel(dinv_ref, a0_ref, a1_ref, y_ref, b_ref, w_ref, o_ref):
    dinv = dinv_ref[...]
    h = jnp.maximum(dinv * (a0_ref[0] + a1_ref[0] + y_ref[...])
                    + b_ref[...], 0.0)
    o_ref[...] = dinv * jnp.dot(h, w_ref[...],
                                preferred_element_type=jnp.float32)


def _tc_head_kernel(dinv_ref, a0_ref, a1_ref, y_ref, b_ref, wl_ref, bl_ref,
                    o_ref):
    dinv = dinv_ref[...]
    h = jnp.maximum(dinv * (a0_ref[0] + a1_ref[0] + y_ref[...])
                    + b_ref[...], 0.0)
    o_ref[...] = jnp.dot(h, wl_ref[...],
                         preferred_element_type=jnp.float32) + bl_ref[...]


def _tc_mm(x, w):
    return pl.pallas_call(
        _tc_mm_kernel,
        grid=(GRID,),
        in_specs=[
            pl.BlockSpec((BM, D), lambda i: (i, 0)),
            pl.BlockSpec((D, H), lambda i: (0, 0)),
        ],
        out_specs=pl.BlockSpec((BM, H), lambda i: (i, 0)),
        out_shape=jax.ShapeDtypeStruct((N, H), jnp.float32),
    )(x, w)


def _tc_scale(degp, z):
    return pl.pallas_call(
        _tc_scale_kernel,
        grid=(GRID,),
        in_specs=[
            pl.BlockSpec((2, BM), lambda i: (0, i)),
            pl.BlockSpec((BM, H), lambda i: (i, 0)),
        ],
        out_specs=[
            pl.BlockSpec((BM, H), lambda i: (i, 0)),
            pl.BlockSpec((BM, 1), lambda i: (i, 0)),
        ],
        out_shape=[
            jax.ShapeDtypeStruct((N, H), jnp.float32),
            jax.ShapeDtypeStruct((N, 1), jnp.float32),
        ],
    )(degp, z)


def _acc_specs():
    return [
        pl.BlockSpec((1, BM, H), lambda i: (0, i, 0)),
        pl.BlockSpec((1, BM, H), lambda i: (1, i, 0)),
    ]


def _tc_layer(dinv, acc, y, b, w):
    return pl.pallas_call(
        _tc_layer_kernel,
        grid=(GRID,),
        in_specs=[pl.BlockSpec((BM, 1), lambda i: (i, 0))] + _acc_specs() + [
            pl.BlockSpec((BM, H), lambda i: (i, 0)),
            pl.BlockSpec((1, H), lambda i: (0, 0)),
            pl.BlockSpec((H, H), lambda i: (0, 0)),
        ],
        out_specs=pl.BlockSpec((BM, H), lambda i: (i, 0)),
        out_shape=jax.ShapeDtypeStruct((N, H), jnp.float32),
    )(dinv, acc, acc, y, b.reshape(1, H), w)


def _tc_head(dinv, acc, y, b, wl, bl):
    no = wl.shape[1]
    return pl.pallas_call(
        _tc_head_kernel,
        grid=(GRID,),
        in_specs=[pl.BlockSpec((BM, 1), lambda i: (i, 0))] + _acc_specs() + [
            pl.BlockSpec((BM, H), lambda i: (i, 0)),
            pl.BlockSpec((1, H), lambda i: (0, 0)),
            pl.BlockSpec((H, no), lambda i: (0, 0)),
            pl.BlockSpec((1, no), lambda i: (0, 0)),
        ],
        out_specs=pl.BlockSpec((BM, no), lambda i: (i, 0)),
        out_shape=jax.ShapeDtypeStruct((N, no), jnp.float32),
    )(dinv, acc, acc, y, b.reshape(1, H), wl, bl.reshape(1, no))


def kernel(x, edge_index, edge_attr, W1, b1, W2, b2, Wl, bl):
    src = edge_index[0]
    dst = edge_index[1]
    pad = EPAD - E
    ar = jnp.arange(pad, dtype=jnp.int32)
    srcp = jnp.concatenate([src, ar % N])
    srcp = srcp.reshape(NC, NS, CHUNKS, K)
    dstp = jnp.concatenate([dst, TRASH + ar % (NPAD - TRASH)])
    dstp = dstp.reshape(NC, NS, CHUNKS, K)
    pairs = jnp.stack([srcp, dstp], axis=3)
    pairs = pairs.reshape(NC, NS, NG, G, 2, K)

    degp = _sc_deg(dstp)
    z1 = _tc_mm(x, W1)
    y1, dinv = _tc_scale(degp, z1)
    acc1 = _sc_agg(y1, pairs)
    y2 = _tc_layer(dinv, acc1, y1, b1, W2)
    acc2 = _sc_agg(y2, pairs)
    out = _tc_head(dinv, acc2, y2, b2, Wl, bl)
    return out
